# baseline (speedup 1.0000x reference)
import functools

import jax
import jax.numpy as jnp
from jax import lax
from jax.experimental import pallas as pl
from jax.experimental.pallas import tpu as pltpu
from jax.experimental.pallas import tpu_sc as plsc

_D = 128
_NC = 2
_NS = 16
_NW = _NC * _NS
_BW = 128
_NB = 5
_LA = 4


@functools.lru_cache(maxsize=None)
def _build(batch, seq):
    assert batch == _NW * _BW
    n_chunks = seq
    assert n_chunks % _NB == 0 and n_chunks // _NB >= 2
    n_groups = n_chunks // _NB

    @functools.partial(
        pl.kernel,
        out_type=jax.ShapeDtypeStruct((seq, batch, _D), jnp.float32),
        mesh=plsc.VectorSubcoreMesh(core_axis_name="c", subcore_axis_name="s"),
        scratch_types=[
            pltpu.VMEM((seq, _BW), jnp.int32),
            [pltpu.VMEM((_BW, _D), jnp.float32) for _ in range(_NB)],
            [pltpu.SemaphoreType.DMA for _ in range(_NB)],
            [pltpu.SemaphoreType.DMA for _ in range(_NB)],
        ],
    )
    def emb(table_hbm, idx_hbm, out_hbm, idx_v, bufs, gsems, osems):
        wid = lax.axis_index("s") * _NC + lax.axis_index("c")
        base = wid * _BW
        pltpu.sync_copy(idx_hbm.at[:, pl.ds(base, _BW)], idx_v)

        def start_gather(j, b):
            pltpu.async_copy(table_hbm.at[idx_v.at[j]], bufs[b], gsems[b])

        def wait_gather(j, b):
            pltpu.make_async_copy(table_hbm.at[idx_v.at[j]], bufs[b],
                                  gsems[b]).wait()

        def start_out(j, b):
            pltpu.async_copy(bufs[b], out_hbm.at[j, pl.ds(base, _BW)], osems[b])

        def wait_out(b):
            pltpu.make_async_copy(bufs[b], out_hbm.at[0, pl.ds(0, _BW)],
                                  osems[b]).wait()

        for b in range(_LA):
            start_gather(b, b)

        for b in range(_NB):
            jn = b + _LA
            if jn < n_chunks:
                bb = jn % _NB
                if jn >= _NB:
                    wait_out(bb)
                start_gather(jn, bb)
            wait_gather(b, b)
            start_out(b, b)

        def group(g, carry):
            j0 = g * _NB
            for b in range(_NB):
                j = j0 + b
                bb = (b + _LA) % _NB
                wait_out(bb)
                start_gather(j + _LA, bb)
                wait_gather(j, b)
                start_out(j, b)
            return carry

        lax.fori_loop(1, n_groups - 1, group, 0)

        j0 = (n_groups - 1) * _NB
        for b in range(_NB):
            j = j0 + b
            jn = j + _LA
            if jn < n_chunks:
                bb = jn % _NB
                wait_out(bb)
                start_gather(jn, bb)
            wait_gather(j, b)
            start_out(j, b)

        for b in range(_NB):
            wait_out(b)

    return emb


def _tc_copy_body(t_ref, o_ref):
    o_ref[...] = t_ref[...]


@functools.lru_cache(maxsize=None)
def _build_table_copy(vocab):
    blk = 4000
    assert vocab % blk == 0
    return pl.pallas_call(
        _tc_copy_body,
        grid=(vocab // blk,),
        in_specs=[pl.BlockSpec((blk, _D), lambda i: (i, 0))],
        out_specs=pl.BlockSpec((blk, _D), lambda i: (i, 0)),
        out_shape=jax.ShapeDtypeStruct((vocab, _D), jnp.float32),
    )


def kernel(inputs, embedding_table):
    b, s = inputs.shape
    idx_t = inputs.astype(jnp.int32).T
    out = _build(b, s)(embedding_table, idx_t)
    table_out = _build_table_copy(embedding_table.shape[0])(embedding_table)
    return out.transpose(1, 0, 2), table_out


# speedup vs baseline: 1.0052x; 1.0052x over previous
import functools

import jax
import jax.numpy as jnp
from jax import lax
from jax.experimental import pallas as pl
from jax.experimental.pallas import tpu as pltpu
from jax.experimental.pallas import tpu_sc as plsc

_D = 128
_NC = 2
_NS = 16
_NW = _NC * _NS
_BW = 128
_NB = 5
_LA = 3


@functools.lru_cache(maxsize=None)
def _build(batch, seq):
    assert batch == _NW * _BW
    n_chunks = seq
    assert n_chunks % _NB == 0 and n_chunks // _NB >= 2
    n_groups = n_chunks // _NB

    @functools.partial(
        pl.kernel,
        out_type=jax.ShapeDtypeStruct((seq, batch, _D), jnp.float32),
        mesh=plsc.VectorSubcoreMesh(core_axis_name="c", subcore_axis_name="s"),
        scratch_types=[
            pltpu.VMEM((seq, _BW), jnp.int32),
            [pltpu.VMEM((_BW, _D), jnp.float32) for _ in range(_NB)],
            [pltpu.SemaphoreType.DMA for _ in range(_NB)],
            [pltpu.SemaphoreType.DMA for _ in range(_NB)],
        ],
    )
    def emb(table_hbm, idx_hbm, out_hbm, idx_v, bufs, gsems, osems):
        wid = lax.axis_index("s") * _NC + lax.axis_index("c")
        base = wid * _BW
        pltpu.sync_copy(idx_hbm.at[:, pl.ds(base, _BW)], idx_v)

        def start_gather(j, b):
            pltpu.async_copy(table_hbm.at[idx_v.at[j]], bufs[b], gsems[b])

        def wait_gather(j, b):
            pltpu.make_async_copy(table_hbm.at[idx_v.at[j]], bufs[b],
                                  gsems[b]).wait()

        def start_out(j, b):
            pltpu.async_copy(bufs[b], out_hbm.at[j, pl.ds(base, _BW)], osems[b])

        def wait_out(b):
            pltpu.make_async_copy(bufs[b], out_hbm.at[0, pl.ds(0, _BW)],
                                  osems[b]).wait()

        for b in range(_LA):
            start_gather(b, b)

        for b in range(_NB):
            jn = b + _LA
            if jn < n_chunks:
                bb = jn % _NB
                if jn >= _NB:
                    wait_out(bb)
                start_gather(jn, bb)
            wait_gather(b, b)
            start_out(b, b)

        def group(g, carry):
            j0 = g * _NB
            for b in range(_NB):
                j = j0 + b
                bb = (b + _LA) % _NB
                wait_out(bb)
                start_gather(j + _LA, bb)
                wait_gather(j, b)
                start_out(j, b)
            return carry

        lax.fori_loop(1, n_groups - 1, group, 0)

        j0 = (n_groups - 1) * _NB
        for b in range(_NB):
            j = j0 + b
            jn = j + _LA
            if jn < n_chunks:
                bb = jn % _NB
                wait_out(bb)
                start_gather(jn, bb)
            wait_gather(j, b)
            start_out(j, b)

        for b in range(_NB):
            wait_out(b)

    return emb


def _tc_copy_body(t_ref, o_ref):
    o_ref[...] = t_ref[...]


@functools.lru_cache(maxsize=None)
def _build_table_copy(vocab):
    blk = 4000
    assert vocab % blk == 0
    return pl.pallas_call(
        _tc_copy_body,
        grid=(vocab // blk,),
        in_specs=[pl.BlockSpec((blk, _D), lambda i: (i, 0))],
        out_specs=pl.BlockSpec((blk, _D), lambda i: (i, 0)),
        out_shape=jax.ShapeDtypeStruct((vocab, _D), jnp.float32),
    )


def kernel(inputs, embedding_table):
    b, s = inputs.shape
    idx_t = inputs.astype(jnp.int32).T
    out = _build(b, s)(embedding_table, idx_t)
    table_out = _build_table_copy(embedding_table.shape[0])(embedding_table)
    return out.transpose(1, 0, 2), table_out
